# baseline (speedup 1.0000x reference)
import functools

import jax
import jax.numpy as jnp
from jax import lax
from jax.experimental import pallas as pl
from jax.experimental.pallas import tpu as pltpu
from jax.experimental.pallas import tpu_sc as plsc

B = 20
NPG = 500
NP = 512
EPG = 16000
E = B * EPG
K = 100
KP = 104
N = B * NPG
EPS = 1e-5
FLAT = NP * NP
TILE_SLICE = FLAT // 16
EPT = EPG // 16
GPC = B // 2

_INTERP = False
_HI = jax.lax.Precision.HIGHEST


def _dot(a, b, dims, prec=None):
    return jax.lax.dot_general(a, b, (dims, ((), ())),
                               preferred_element_type=jnp.float32,
                               precision=prec)


def _sc_body(src_hbm, dst_hbm, ea_hbm, out_ea, out_cnt,
             sh_ea, sh_cnt, src_v, dst_v, ea_v,
             idx_v, vea_v, vcnt_v, zero_v, sem):
    c = lax.axis_index("c")
    s = lax.axis_index("s")
    lane = lax.broadcasted_iota(jnp.int32, (16,), 0)

    def zfill(j, _):
        zero_v[pl.ds(j * 16, 16)] = jnp.zeros((16,), jnp.float32)
        return 0
    lax.fori_loop(0, TILE_SLICE // 16, zfill, 0)

    def per_graph(i, _):
        b = c * GPC + i
        off = b * NPG
        pltpu.sync_copy(zero_v, sh_ea.at[pl.ds(s * TILE_SLICE, TILE_SLICE)])
        pltpu.sync_copy(zero_v, sh_cnt.at[pl.ds(s * TILE_SLICE, TILE_SLICE)])
        plsc.subcore_barrier()
        base = b * EPG + s * EPT
        pltpu.sync_copy(src_hbm.at[pl.ds(base, EPT)], src_v.at[pl.ds(0, EPT)])
        pltpu.sync_copy(dst_hbm.at[pl.ds(base, EPT)], dst_v.at[pl.ds(0, EPT)])
        pltpu.sync_copy(ea_hbm.at[pl.ds(base, EPT)], ea_v.at[pl.ds(0, EPT)])
        descs = []
        for r in range(8):
            def chunk(j, _, r=r):
                e0 = r * 128 + j * 16
                valid = (e0 + lane) < EPT
                s16 = src_v[pl.ds(e0, 16)] - off
                d16 = dst_v[pl.ds(e0, 16)] - off
                idx_v[r, pl.ds(j * 16, 16)] = jnp.where(valid, d16 * NP + s16, 0)
                vea_v[r, pl.ds(j * 16, 16)] = jnp.where(valid, ea_v[pl.ds(e0, 16)], 0.0)
                vcnt_v[r, pl.ds(j * 16, 16)] = jnp.where(valid, 1.0, 0.0)
                return 0
            lax.fori_loop(0, 8, chunk, 0)
            descs.append(pltpu.async_copy(vea_v.at[r], sh_ea.at[idx_v.at[r]], sem, add=True))
            descs.append(pltpu.async_copy(vcnt_v.at[r], sh_cnt.at[idx_v.at[r]], sem, add=True))
        for d in descs:
            d.wait()
        plsc.subcore_barrier()
        pltpu.sync_copy(sh_ea.at[pl.ds(s * TILE_SLICE, TILE_SLICE)],
                        out_ea.at[b, pl.ds(s * TILE_SLICE, TILE_SLICE)])
        pltpu.sync_copy(sh_cnt.at[pl.ds(s * TILE_SLICE, TILE_SLICE)],
                        out_cnt.at[b, pl.ds(s * TILE_SLICE, TILE_SLICE)])
        return 0

    lax.fori_loop(0, GPC, per_graph, 0)


def _build_adj(src, dst, ea):
    mesh = plsc.VectorSubcoreMesh(core_axis_name="c", subcore_axis_name="s")
    fn = functools.partial(
        pl.kernel,
        out_type=[jax.ShapeDtypeStruct((B, FLAT), jnp.float32),
                  jax.ShapeDtypeStruct((B, FLAT), jnp.float32)],
        mesh=mesh,
        scratch_types=[
            pltpu.VMEM_SHARED((FLAT,), jnp.float32),
            pltpu.VMEM_SHARED((FLAT,), jnp.float32),
            pltpu.VMEM((1024,), jnp.int32),
            pltpu.VMEM((1024,), jnp.int32),
            pltpu.VMEM((1024,), jnp.float32),
            pltpu.VMEM((8, 128), jnp.int32),
            pltpu.VMEM((8, 128), jnp.float32),
            pltpu.VMEM((8, 128), jnp.float32),
            pltpu.VMEM((TILE_SLICE,), jnp.float32),
            pltpu.SemaphoreType.DMA,
        ],
    )(_sc_body)
    ate, atc = fn(src, dst, ea)
    return ate.reshape(B, NP, NP), atc.reshape(B, NP, NP)


def _gcn_pair_body(ate_ref, atc_ref, hc_ref, hp_ref,
                   ac_ref, cc_ref, wc_ref, bc_ref,
                   ap_ref, cp_ref, wp_ref, bp_ref,
                   oc_ref, op_ref, sc_ref, sp_ref):
    row_ok = jax.lax.broadcasted_iota(jnp.int32, (NP, 1), 0) < NPG

    def one_path(at, h, a, c, w, bias, o_ref, s_ref):
        deg = jnp.sum(at, axis=1, keepdims=True) + 1.0
        dinv = jax.lax.rsqrt(deg)
        hw = _dot(h * a + c, w, ((1,), (0,)))
        v = _dot(at, dinv * hw, ((1,), (0,)), _HI)
        out = dinv * v + (dinv * dinv) * hw + bias
        out = jnp.where(row_ok, out, 0.0)
        o_ref[0] = out
        st = jnp.concatenate([jnp.sum(out, axis=0, keepdims=True),
                              jnp.sum(out * out, axis=0, keepdims=True),
                              jnp.zeros((6, out.shape[1]), jnp.float32)], axis=0)

        @pl.when(pl.program_id(0) == 0)
        def _():
            s_ref[...] = st

        @pl.when(pl.program_id(0) != 0)
        def _():
            s_ref[...] += st

    one_path(ate_ref[0], hc_ref[0], ac_ref[...], cc_ref[...],
             wc_ref[...], bc_ref[...], oc_ref, sc_ref)
    one_path(atc_ref[0], hp_ref[0], ap_ref[...], cp_ref[...],
             wp_ref[...], bp_ref[...], op_ref, sp_ref)


def _gcn_pair(ate, atc, hc, hp, ac, cc, wc, bc, ap, cp, wp, bp):
    co_c, co_p = wc.shape[1], wp.shape[1]
    blk3 = lambda d2, d3: pl.BlockSpec((1, d2, d3), lambda b: (b, 0, 0))
    full = lambda a: pl.BlockSpec(a.shape, lambda b: (0,) * a.ndim)
    args = (ate, atc, hc, hp, ac, cc, wc, bc, ap, cp, wp, bp)
    return pl.pallas_call(
        _gcn_pair_body,
        grid=(B,),
        in_specs=[blk3(NP, NP), blk3(NP, NP), blk3(NP, hc.shape[2]), blk3(NP, hp.shape[2])]
                 + [full(a) for a in args[4:]],
        out_specs=[blk3(NP, co_c), blk3(NP, co_p),
                   pl.BlockSpec((8, co_c), lambda b: (0, 0)),
                   pl.BlockSpec((8, co_p), lambda b: (0, 0))],
        out_shape=[jax.ShapeDtypeStruct((B, NP, co_c), jnp.float32),
                   jax.ShapeDtypeStruct((B, NP, co_p), jnp.float32),
                   jax.ShapeDtypeStruct((8, co_c), jnp.float32),
                   jax.ShapeDtypeStruct((8, co_p), jnp.float32)],
        interpret=_INTERP,
    )(*args)


def _diffpool_body(ate_ref, x11_ref, x12_ref, x13_ref, s11_ref, s12_ref, s13_ref,
                   wfc_ref, bfc_ref,
                   a11_ref, c11_ref, a12_ref, c12_ref, a13_ref, c13_ref,
                   b11_ref, d11_ref, b12_ref, d12_ref, b13_ref, d13_ref,
                   px_ref, padj_ref, x1o_ref):
    row_ok = jax.lax.broadcasted_iota(jnp.int32, (NP, 1), 0) < NPG
    s_cat = jnp.concatenate([s11_ref[0] * b11_ref[...] + d11_ref[...],
                             s12_ref[0] * b12_ref[...] + d12_ref[...],
                             s13_ref[0] * b13_ref[...] + d13_ref[...]], axis=1)
    s1 = _dot(s_cat, wfc_ref[...], ((1,), (0,))) + bfc_ref[...]
    mx = jnp.max(s1, axis=1, keepdims=True)
    e = jnp.exp(s1 - mx)
    s = e / jnp.sum(e, axis=1, keepdims=True)
    s = jnp.where(row_ok, s, 0.0)
    s_pad = jnp.concatenate([s, jnp.zeros((NP, KP - K), jnp.float32)], axis=1)

    x13bn = x13_ref[0] * a13_ref[...] + c13_ref[...]
    px_ref[0] = _dot(s_pad, x13bn, ((0,), (0,)))
    tmpd = _dot(ate_ref[0], s_pad, ((0,), (0,)))
    padj_ref[0] = _dot(s_pad, tmpd, ((0,), (0,)))

    x1cat = jnp.concatenate([x11_ref[0] * a11_ref[...] + c11_ref[...],
                             x12_ref[0] * a12_ref[...] + c12_ref[...],
                             x13bn], axis=1)
    x1cat = jnp.where(row_ok, x1cat, -1e30)
    x1o_ref[0, 0] = jnp.max(x1cat, axis=0)


def _diffpool(ate, x11, x12, x13, s11, s12, s13, wfc, bfc,
              a11, c11, a12, c12, a13, c13, b11, d11, b12, d12, b13, d13):
    blk3 = lambda d2, d3: pl.BlockSpec((1, d2, d3), lambda b: (b, 0, 0))
    full = lambda a: pl.BlockSpec(a.shape, lambda b: (0,) * a.ndim)
    args = (ate, x11, x12, x13, s11, s12, s13, wfc, bfc,
            a11, c11, a12, c12, a13, c13, b11, d11, b12, d12, b13, d13)
    return pl.pallas_call(
        _diffpool_body,
        grid=(B,),
        in_specs=[blk3(NP, NP)] + [blk3(NP, 30)] * 4 + [blk3(NP, 30), blk3(NP, K)]
                 + [full(a) for a in args[7:]],
        out_specs=[blk3(KP, 30), blk3(KP, KP), pl.BlockSpec((1, 1, 90), lambda b: (b, 0, 0))],
        out_shape=[jax.ShapeDtypeStruct((B, KP, 30), jnp.float32),
                   jax.ShapeDtypeStruct((B, KP, KP), jnp.float32),
                   jax.ShapeDtypeStruct((B, 1, 90), jnp.float32)],
        interpret=_INTERP,
    )(*args)


def _dense_gcn_body(adj_ref, h_ref, a_ref, c_ref, w_ref, b_ref, o_ref, st_ref):
    row_ok = jax.lax.broadcasted_iota(jnp.int32, (KP, 1), 0) < K
    ri = jax.lax.broadcasted_iota(jnp.int32, (KP, KP), 0)
    ci = jax.lax.broadcasted_iota(jnp.int32, (KP, KP), 1)
    eye = jnp.where(ri == ci, 1.0, 0.0)
    adj = adj_ref[0]
    diag = jnp.sum(adj * eye, axis=0, keepdims=True)
    a2 = adj + eye * jnp.where(diag == 0.0, 1.0, 0.0)
    ones = jnp.ones((KP, 1), jnp.float32)
    deg = _dot(a2, ones, ((0,), (0,)), _HI)
    dinv = jnp.where(deg > 0.0, jax.lax.rsqrt(jnp.where(deg > 0.0, deg, 1.0)), 0.0)
    hw = _dot(h_ref[0] * a_ref[...] + c_ref[...], w_ref[...], ((1,), (0,)))
    v = _dot(a2, dinv * hw, ((0,), (0,)))
    out = jnp.where(row_ok, dinv * v + b_ref[...], 0.0)
    o_ref[0] = out
    st = jnp.concatenate([jnp.sum(out, axis=0, keepdims=True),
                          jnp.sum(out * out, axis=0, keepdims=True),
                          jnp.zeros((6, out.shape[1]), jnp.float32)], axis=0)

    @pl.when(pl.program_id(0) == 0)
    def _():
        st_ref[...] = st

    @pl.when(pl.program_id(0) != 0)
    def _():
        st_ref[...] += st


def _dense_gcn(adj, h, a, c, w, bias):
    co = w.shape[1]
    blk3 = lambda d2, d3: pl.BlockSpec((1, d2, d3), lambda b: (b, 0, 0))
    full = lambda x: pl.BlockSpec(x.shape, lambda b: (0,) * x.ndim)
    return pl.pallas_call(
        _dense_gcn_body,
        grid=(B,),
        in_specs=[blk3(KP, KP), blk3(KP, h.shape[2]), full(a), full(c), full(w), full(bias)],
        out_specs=[blk3(KP, co), pl.BlockSpec((8, co), lambda b: (0, 0))],
        out_shape=[jax.ShapeDtypeStruct((B, KP, co), jnp.float32),
                   jax.ShapeDtypeStruct((8, co), jnp.float32)],
        interpret=_INTERP,
    )(adj, h, a, c, w, bias)


def _final_body(x1o_ref, x21_ref, x22_ref, x23_ref,
                a21_ref, c21_ref, a22_ref, c22_ref, a23_ref, c23_ref,
                w1_ref, b1_ref, w2_ref, b2_ref, o_ref):
    row_ok = jax.lax.broadcasted_iota(jnp.int32, (KP, 1), 0) < K
    x2cat = jnp.concatenate([x21_ref[0] * a21_ref[...] + c21_ref[...],
                             x22_ref[0] * a22_ref[...] + c22_ref[...],
                             x23_ref[0] * a23_ref[...] + c23_ref[...]], axis=1)
    x2cat = jnp.where(row_ok, x2cat, -1e30)
    x2row = jnp.max(x2cat, axis=0)
    conv = jnp.concatenate([x1o_ref[0, 0], x2row], axis=0).reshape(1, 180)
    h = jax.nn.relu(_dot(conv, w1_ref[...], ((1,), (0,))) + b1_ref[...])
    o_ref[0, 0] = (_dot(h, w2_ref[...], ((1,), (0,))) + b2_ref[...])[0]


def _final(x1o, x21, x22, x23, a21, c21, a22, c22, a23, c23, w1, b1, w2, b2):
    blk3 = lambda d2, d3: pl.BlockSpec((1, d2, d3), lambda b: (b, 0, 0))
    full = lambda a: pl.BlockSpec(a.shape, lambda b: (0,) * a.ndim)
    args = (x1o, x21, x22, x23, a21, c21, a22, c22, a23, c23, w1, b1, w2, b2)
    return pl.pallas_call(
        _final_body,
        grid=(B,),
        in_specs=[blk3(1, 90), blk3(KP, 30), blk3(KP, 30), blk3(KP, 30)]
                 + [full(a) for a in args[4:]],
        out_specs=pl.BlockSpec((1, 1, 6), lambda b: (b, 0, 0)),
        out_shape=jax.ShapeDtypeStruct((B, 1, 6), jnp.float32),
        interpret=_INTERP,
    )(*args)


def _stats_fold(st, n_rows, bn):
    m = st[0] / n_rows
    var = st[1] / n_rows - m * m
    a = bn["g"] / jnp.sqrt(var + EPS)
    c = bn["be"] - m * a
    return a.reshape(1, -1), c.reshape(1, -1)


def kernel(x, edge_attr, params, edge_index):
    p = params
    src = edge_index[0].astype(jnp.int32)
    dst = edge_index[1].astype(jnp.int32)
    ate, atc = _build_adj(src, dst, edge_attr)

    xb = jnp.pad(x.reshape(B, NPG, 3), ((0, 0), (0, NP - NPG), (0, 5)))
    w1c = jnp.pad(p["conv11"]["W"], ((0, 5), (0, 0)))
    w1p = jnp.pad(p["pool_conv11"]["W"], ((0, 5), (0, 0)))
    one8 = jnp.ones((1, 8), jnp.float32)
    zero8 = jnp.zeros((1, 8), jnp.float32)
    one30 = jnp.ones((1, 30), jnp.float32)
    zero30 = jnp.zeros((1, 30), jnp.float32)
    rb = lambda v: v.reshape(1, -1)

    x11p, s11p, st11c, st11p = _gcn_pair(ate, atc, xb, xb,
                                         one8, zero8, w1c, rb(p["conv11"]["b"]),
                                         one8, zero8, w1p, rb(p["pool_conv11"]["b"]))
    a11, c11 = _stats_fold(st11c, N, p["norm11"])
    b11, d11 = _stats_fold(st11p, N, p["norm_p11"])
    x12p, s12p, st12c, st12p = _gcn_pair(ate, atc, x11p, s11p,
                                         a11, c11, p["conv12"]["W"], rb(p["conv12"]["b"]),
                                         b11, d11, p["pool_conv12"]["W"], rb(p["pool_conv12"]["b"]))
    a12, c12 = _stats_fold(st12c, N, p["norm12"])
    b12, d12 = _stats_fold(st12p, N, p["norm_p12"])
    x13p, s13p, st13c, st13p = _gcn_pair(ate, atc, x12p, s12p,
                                         a12, c12, p["conv13"]["W"], rb(p["conv13"]["b"]),
                                         b12, d12, p["pool_conv13"]["W"], rb(p["pool_conv13"]["b"]))
    a13, c13 = _stats_fold(st13c, N, p["norm13"])
    b13, d13 = _stats_fold(st13p, N, p["norm_p13"])

    p1x, p1adj, x1o = _diffpool(ate, x11p, x12p, x13p, s11p, s12p, s13p,
                                p["pool_fc"]["W"], rb(p["pool_fc"]["b"]),
                                a11, c11, a12, c12, a13, c13,
                                b11, d11, b12, d12, b13, d13)

    x21p, st21 = _dense_gcn(p1adj, p1x, one30, zero30, p["conv21"]["W"], rb(p["conv21"]["b"]))
    a21, c21 = _stats_fold(st21, B * K, p["norm21"])
    x22p, st22 = _dense_gcn(p1adj, x21p, a21, c21, p["conv22"]["W"], rb(p["conv22"]["b"]))
    a22, c22 = _stats_fold(st22, B * K, p["norm22"])
    x23p, st23 = _dense_gcn(p1adj, x22p, a22, c22, p["conv23"]["W"], rb(p["conv23"]["b"]))
    a23, c23 = _stats_fold(st23, B * K, p["norm23"])

    out = _final(x1o, x21p, x22p, x23p, a21, c21, a22, c22, a23, c23,
                 p["fc1"]["W"], rb(p["fc1"]["b"]),
                 p["fc2"]["W"], rb(p["fc2"]["b"]))
    reg = jnp.zeros((1,), x.dtype)
    return (out.reshape(B, 6), reg)


# speedup vs baseline: 1.3562x; 1.3562x over previous
import functools

import jax
import jax.numpy as jnp
from jax import lax
from jax.experimental import pallas as pl
from jax.experimental.pallas import tpu as pltpu
from jax.experimental.pallas import tpu_sc as plsc

B = 20
NPG = 500
NP = 512
EPG = 16000
E = B * EPG
K = 100
KP = 104
N = B * NPG
EPS = 1e-5
FLAT = NP * NP
TILE_SLICE = FLAT // 16
EPT = EPG // 16
GPC = B // 2

_INTERP = False
_HI = jax.lax.Precision.HIGHEST


def _dot(a, b, dims, prec=None):
    return jax.lax.dot_general(a, b, (dims, ((), ())),
                               preferred_element_type=jnp.float32,
                               precision=prec)


def _sc_body(src_hbm, dst_hbm, ea_hbm, out_ea, out_cnt,
             sh_ea, sh_cnt, src_v, dst_v, ea_v,
             idx_v, vea_v, vcnt_v, zero_v, sem):
    c = lax.axis_index("c")
    s = lax.axis_index("s")
    lane = lax.broadcasted_iota(jnp.int32, (16,), 0)

    def zfill(j, _):
        zero_v[pl.ds(j * 16, 16)] = jnp.zeros((16,), jnp.float32)
        return 0
    lax.fori_loop(0, TILE_SLICE // 16, zfill, 0)

    def per_graph(i, _):
        b = c * GPC + i
        off = b * NPG
        pltpu.sync_copy(zero_v, sh_ea.at[pl.ds(s * TILE_SLICE, TILE_SLICE)])
        pltpu.sync_copy(zero_v, sh_cnt.at[pl.ds(s * TILE_SLICE, TILE_SLICE)])
        plsc.subcore_barrier()
        base = b * EPG + s * EPT
        pltpu.sync_copy(src_hbm.at[pl.ds(base, EPT)], src_v.at[pl.ds(0, EPT)])
        pltpu.sync_copy(dst_hbm.at[pl.ds(base, EPT)], dst_v.at[pl.ds(0, EPT)])
        pltpu.sync_copy(ea_hbm.at[pl.ds(base, EPT)], ea_v.at[pl.ds(0, EPT)])
        descs = []
        for r in range(8):
            def chunk(j, _, r=r):
                e0 = r * 128 + j * 16
                valid = (e0 + lane) < EPT
                s16 = src_v[pl.ds(e0, 16)] - off
                d16 = dst_v[pl.ds(e0, 16)] - off
                idx_v[r, pl.ds(j * 16, 16)] = jnp.where(valid, d16 * NP + s16, 0)
                vea_v[r, pl.ds(j * 16, 16)] = jnp.where(valid, ea_v[pl.ds(e0, 16)], 0.0)
                vcnt_v[r, pl.ds(j * 16, 16)] = jnp.where(valid, 1.0, 0.0)
                return 0
            lax.fori_loop(0, 8, chunk, 0)
            descs.append(pltpu.async_copy(vea_v.at[r], sh_ea.at[idx_v.at[r]], sem, add=True))
            descs.append(pltpu.async_copy(vcnt_v.at[r], sh_cnt.at[idx_v.at[r]], sem, add=True))
        for d in descs:
            d.wait()
        plsc.subcore_barrier()
        outs = []
        for k in range(32):
            row = s * 32 + k
            outs.append(pltpu.async_copy(sh_ea.at[pl.ds(row * NP, NP)],
                                         out_ea.at[b, row, :], sem))
            outs.append(pltpu.async_copy(sh_cnt.at[pl.ds(row * NP, NP)],
                                         out_cnt.at[b, row, :], sem))
        for d in outs:
            d.wait()
        return 0

    lax.fori_loop(0, GPC, per_graph, 0)


def _build_adj(src, dst, ea):
    mesh = plsc.VectorSubcoreMesh(core_axis_name="c", subcore_axis_name="s")
    fn = functools.partial(
        pl.kernel,
        out_type=[jax.ShapeDtypeStruct((B, NP, NP), jnp.float32),
                  jax.ShapeDtypeStruct((B, NP, NP), jnp.float32)],
        mesh=mesh,
        scratch_types=[
            pltpu.VMEM_SHARED((FLAT,), jnp.float32),
            pltpu.VMEM_SHARED((FLAT,), jnp.float32),
            pltpu.VMEM((1024,), jnp.int32),
            pltpu.VMEM((1024,), jnp.int32),
            pltpu.VMEM((1024,), jnp.float32),
            pltpu.VMEM((8, 128), jnp.int32),
            pltpu.VMEM((8, 128), jnp.float32),
            pltpu.VMEM((8, 128), jnp.float32),
            pltpu.VMEM((TILE_SLICE,), jnp.float32),
            pltpu.SemaphoreType.DMA,
        ],
    )(_sc_body)
    return fn(src, dst, ea)


def _gcn_pair_body(ate_ref, atc_ref, hc_ref, hp_ref,
                   ac_ref, cc_ref, wc_ref, bc_ref,
                   ap_ref, cp_ref, wp_ref, bp_ref,
                   oc_ref, op_ref):
    row_ok = jax.lax.broadcasted_iota(jnp.int32, (NP, 1), 0) < NPG

    def one_path(at, h, a, c, w, bias):
        deg = jnp.sum(at, axis=1, keepdims=True) + 1.0
        dinv = jax.lax.rsqrt(deg)
        hw = _dot(h * a + c, w, ((1,), (0,)))
        v = _dot(at, dinv * hw, ((1,), (0,)), _HI)
        out = dinv * v + (dinv * dinv) * hw + bias
        return jnp.where(row_ok, out, 0.0)

    oc_ref[0] = one_path(ate_ref[0], hc_ref[0], ac_ref[...], cc_ref[...],
                         wc_ref[...], bc_ref[...])
    op_ref[0] = one_path(atc_ref[0], hp_ref[0], ap_ref[...], cp_ref[...],
                         wp_ref[...], bp_ref[...])


def _gcn_pair(ate, atc, hc, hp, ac, cc, wc, bc, ap, cp, wp, bp):
    co_c, co_p = wc.shape[1], wp.shape[1]
    blk3 = lambda d2, d3: pl.BlockSpec((1, d2, d3), lambda b: (b, 0, 0))
    full = lambda a: pl.BlockSpec(a.shape, lambda b: (0,) * a.ndim)
    args = (ate, atc, hc, hp, ac, cc, wc, bc, ap, cp, wp, bp)
    return pl.pallas_call(
        _gcn_pair_body,
        grid=(B,),
        in_specs=[blk3(NP, NP), blk3(NP, NP), blk3(NP, hc.shape[2]), blk3(NP, hp.shape[2])]
                 + [full(a) for a in args[4:]],
        out_specs=[blk3(NP, co_c), blk3(NP, co_p)],
        out_shape=[jax.ShapeDtypeStruct((B, NP, co_c), jnp.float32),
                   jax.ShapeDtypeStruct((B, NP, co_p), jnp.float32)],
        interpret=_INTERP,
    )(*args)


def _diffpool_body(ate_ref, x11_ref, x12_ref, x13_ref, s11_ref, s12_ref, s13_ref,
                   wfc_ref, bfc_ref,
                   a11_ref, c11_ref, a12_ref, c12_ref, a13_ref, c13_ref,
                   b11_ref, d11_ref, b12_ref, d12_ref, b13_ref, d13_ref,
                   px_ref, padj_ref, x1o_ref):
    row_ok = jax.lax.broadcasted_iota(jnp.int32, (NP, 1), 0) < NPG
    s_cat = jnp.concatenate([s11_ref[0] * b11_ref[...] + d11_ref[...],
                             s12_ref[0] * b12_ref[...] + d12_ref[...],
                             s13_ref[0] * b13_ref[...] + d13_ref[...]], axis=1)
    s1 = _dot(s_cat, wfc_ref[...], ((1,), (0,))) + bfc_ref[...]
    mx = jnp.max(s1, axis=1, keepdims=True)
    e = jnp.exp(s1 - mx)
    s = e / jnp.sum(e, axis=1, keepdims=True)
    s = jnp.where(row_ok, s, 0.0)
    s_pad = jnp.concatenate([s, jnp.zeros((NP, KP - K), jnp.float32)], axis=1)

    x13bn = x13_ref[0] * a13_ref[...] + c13_ref[...]
    px_ref[0] = _dot(s_pad, x13bn, ((0,), (0,)))
    tmpd = _dot(ate_ref[0], s_pad, ((0,), (0,)))
    padj_ref[0] = _dot(s_pad, tmpd, ((0,), (0,)))

    x1cat = jnp.concatenate([x11_ref[0] * a11_ref[...] + c11_ref[...],
                             x12_ref[0] * a12_ref[...] + c12_ref[...],
                             x13bn], axis=1)
    x1cat = jnp.where(row_ok, x1cat, -1e30)
    x1o_ref[0, 0] = jnp.max(x1cat, axis=0)


def _diffpool(ate, x11, x12, x13, s11, s12, s13, wfc, bfc,
              a11, c11, a12, c12, a13, c13, b11, d11, b12, d12, b13, d13):
    blk3 = lambda d2, d3: pl.BlockSpec((1, d2, d3), lambda b: (b, 0, 0))
    full = lambda a: pl.BlockSpec(a.shape, lambda b: (0,) * a.ndim)
    args = (ate, x11, x12, x13, s11, s12, s13, wfc, bfc,
            a11, c11, a12, c12, a13, c13, b11, d11, b12, d12, b13, d13)
    return pl.pallas_call(
        _diffpool_body,
        grid=(B,),
        in_specs=[blk3(NP, NP)] + [blk3(NP, 30)] * 4 + [blk3(NP, 30), blk3(NP, K)]
                 + [full(a) for a in args[7:]],
        out_specs=[blk3(KP, 30), blk3(KP, KP), pl.BlockSpec((1, 1, 90), lambda b: (b, 0, 0))],
        out_shape=[jax.ShapeDtypeStruct((B, KP, 30), jnp.float32),
                   jax.ShapeDtypeStruct((B, KP, KP), jnp.float32),
                   jax.ShapeDtypeStruct((B, 1, 90), jnp.float32)],
        interpret=_INTERP,
    )(*args)


def _dense_gcn_body(adj_ref, h_ref, a_ref, c_ref, w_ref, b_ref, o_ref):
    row_ok = jax.lax.broadcasted_iota(jnp.int32, (KP, 1), 0) < K
    ri = jax.lax.broadcasted_iota(jnp.int32, (KP, KP), 0)
    ci = jax.lax.broadcasted_iota(jnp.int32, (KP, KP), 1)
    eye = jnp.where(ri == ci, 1.0, 0.0)
    adj = adj_ref[0]
    diag = jnp.sum(adj * eye, axis=0, keepdims=True)
    a2 = adj + eye * jnp.where(diag == 0.0, 1.0, 0.0)
    ones = jnp.ones((KP, 1), jnp.float32)
    deg = _dot(a2, ones, ((0,), (0,)), _HI)
    dinv = jnp.where(deg > 0.0, jax.lax.rsqrt(jnp.where(deg > 0.0, deg, 1.0)), 0.0)
    hw = _dot(h_ref[0] * a_ref[...] + c_ref[...], w_ref[...], ((1,), (0,)))
    v = _dot(a2, dinv * hw, ((0,), (0,)))
    o_ref[0] = jnp.where(row_ok, dinv * v + b_ref[...], 0.0)


def _dense_gcn(adj, h, a, c, w, bias):
    co = w.shape[1]
    blk3 = lambda d2, d3: pl.BlockSpec((1, d2, d3), lambda b: (b, 0, 0))
    full = lambda x: pl.BlockSpec(x.shape, lambda b: (0,) * x.ndim)
    return pl.pallas_call(
        _dense_gcn_body,
        grid=(B,),
        in_specs=[blk3(KP, KP), blk3(KP, h.shape[2]), full(a), full(c), full(w), full(bias)],
        out_specs=blk3(KP, co),
        out_shape=jax.ShapeDtypeStruct((B, KP, co), jnp.float32),
        interpret=_INTERP,
    )(adj, h, a, c, w, bias)


def _final_body(x1o_ref, x21_ref, x22_ref, x23_ref,
                a21_ref, c21_ref, a22_ref, c22_ref, a23_ref, c23_ref,
                w1_ref, b1_ref, w2_ref, b2_ref, o_ref):
    row_ok = jax.lax.broadcasted_iota(jnp.int32, (KP, 1), 0) < K
    x2cat = jnp.concatenate([x21_ref[0] * a21_ref[...] + c21_ref[...],
                             x22_ref[0] * a22_ref[...] + c22_ref[...],
                             x23_ref[0] * a23_ref[...] + c23_ref[...]], axis=1)
    x2cat = jnp.where(row_ok, x2cat, -1e30)
    x2row = jnp.max(x2cat, axis=0)
    conv = jnp.concatenate([x1o_ref[0, 0], x2row], axis=0).reshape(1, 180)
    h = jax.nn.relu(_dot(conv, w1_ref[...], ((1,), (0,))) + b1_ref[...])
    o_ref[0, 0] = (_dot(h, w2_ref[...], ((1,), (0,))) + b2_ref[...])[0]


def _final(x1o, x21, x22, x23, a21, c21, a22, c22, a23, c23, w1, b1, w2, b2):
    blk3 = lambda d2, d3: pl.BlockSpec((1, d2, d3), lambda b: (b, 0, 0))
    full = lambda a: pl.BlockSpec(a.shape, lambda b: (0,) * a.ndim)
    args = (x1o, x21, x22, x23, a21, c21, a22, c22, a23, c23, w1, b1, w2, b2)
    return pl.pallas_call(
        _final_body,
        grid=(B,),
        in_specs=[blk3(1, 90), blk3(KP, 30), blk3(KP, 30), blk3(KP, 30)]
                 + [full(a) for a in args[4:]],
        out_specs=pl.BlockSpec((1, 1, 6), lambda b: (b, 0, 0)),
        out_shape=jax.ShapeDtypeStruct((B, 1, 6), jnp.float32),
        interpret=_INTERP,
    )(*args)


def _stats_fold(h_pre, n_rows, bn):
    s1 = h_pre.sum(axis=(0, 1))
    s2 = (h_pre * h_pre).sum(axis=(0, 1))
    m = s1 / n_rows
    var = s2 / n_rows - m * m
    a = bn["g"] / jnp.sqrt(var + EPS)
    c = bn["be"] - m * a
    return a.reshape(1, -1), c.reshape(1, -1)


def kernel(x, edge_attr, params, edge_index):
    p = params
    src = edge_index[0].astype(jnp.int32)
    dst = edge_index[1].astype(jnp.int32)
    ate, atc = _build_adj(src, dst, edge_attr)

    xb = jnp.pad(x.reshape(B, NPG, 3), ((0, 0), (0, NP - NPG), (0, 5)))
    w1c = jnp.pad(p["conv11"]["W"], ((0, 5), (0, 0)))
    w1p = jnp.pad(p["pool_conv11"]["W"], ((0, 5), (0, 0)))
    one8 = jnp.ones((1, 8), jnp.float32)
    zero8 = jnp.zeros((1, 8), jnp.float32)
    one30 = jnp.ones((1, 30), jnp.float32)
    zero30 = jnp.zeros((1, 30), jnp.float32)
    rb = lambda v: v.reshape(1, -1)

    x11p, s11p = _gcn_pair(ate, atc, xb, xb,
                           one8, zero8, w1c, rb(p["conv11"]["b"]),
                           one8, zero8, w1p, rb(p["pool_conv11"]["b"]))
    a11, c11 = _stats_fold(x11p, N, p["norm11"])
    b11, d11 = _stats_fold(s11p, N, p["norm_p11"])
    x12p, s12p = _gcn_pair(ate, atc, x11p, s11p,
                           a11, c11, p["conv12"]["W"], rb(p["conv12"]["b"]),
                           b11, d11, p["pool_conv12"]["W"], rb(p["pool_conv12"]["b"]))
    a12, c12 = _stats_fold(x12p, N, p["norm12"])
    b12, d12 = _stats_fold(s12p, N, p["norm_p12"])
    x13p, s13p = _gcn_pair(ate, atc, x12p, s12p,
                           a12, c12, p["conv13"]["W"], rb(p["conv13"]["b"]),
                           b12, d12, p["pool_conv13"]["W"], rb(p["pool_conv13"]["b"]))
    a13, c13 = _stats_fold(x13p, N, p["norm13"])
    b13, d13 = _stats_fold(s13p, N, p["norm_p13"])

    p1x, p1adj, x1o = _diffpool(ate, x11p, x12p, x13p, s11p, s12p, s13p,
                                p["pool_fc"]["W"], rb(p["pool_fc"]["b"]),
                                a11, c11, a12, c12, a13, c13,
                                b11, d11, b12, d12, b13, d13)

    x21p = _dense_gcn(p1adj, p1x, one30, zero30, p["conv21"]["W"], rb(p["conv21"]["b"]))
    a21, c21 = _stats_fold(x21p, B * K, p["norm21"])
    x22p = _dense_gcn(p1adj, x21p, a21, c21, p["conv22"]["W"], rb(p["conv22"]["b"]))
    a22, c22 = _stats_fold(x22p, B * K, p["norm22"])
    x23p = _dense_gcn(p1adj, x22p, a22, c22, p["conv23"]["W"], rb(p["conv23"]["b"]))
    a23, c23 = _stats_fold(x23p, B * K, p["norm23"])

    out = _final(x1o, x21p, x22p, x23p, a21, c21, a22, c22, a23, c23,
                 p["fc1"]["W"], rb(p["fc1"]["b"]),
                 p["fc2"]["W"], rb(p["fc2"]["b"]))
    reg = jnp.zeros((1,), x.dtype)
    return (out.reshape(B, 6), reg)


# speedup vs baseline: 1.3649x; 1.0064x over previous
import functools

import jax
import jax.numpy as jnp
from jax import lax
from jax.experimental import pallas as pl
from jax.experimental.pallas import tpu as pltpu
from jax.experimental.pallas import tpu_sc as plsc

B = 20
NPG = 500
NP = 512
EPG = 16000
E = B * EPG
K = 100
KP = 104
N = B * NPG
EPS = 1e-5
FLAT = NP * NP
TILE_SLICE = FLAT // 16
EPT = EPG // 16
GPC = B // 2

_INTERP = False
_HI = jax.lax.Precision.HIGHEST


def _dot(a, b, dims, prec=None):
    return jax.lax.dot_general(a, b, (dims, ((), ())),
                               preferred_element_type=jnp.float32,
                               precision=prec)


def _sc_body(src_hbm, dst_hbm, ea_hbm, out_ea, out_cnt,
             sh_ea, sh_cnt, src_v, dst_v, ea_v,
             idx_v, vea_v, vcnt_v, zero_v, sem):
    c = lax.axis_index("c")
    s = lax.axis_index("s")
    lane = lax.broadcasted_iota(jnp.int32, (16,), 0)

    def zfill(j, _):
        zero_v[pl.ds(j * 16, 16)] = jnp.zeros((16,), jnp.float32)
        return 0
    lax.fori_loop(0, TILE_SLICE // 16, zfill, 0)

    def per_graph(i, _):
        b = c * GPC + i
        off = b * NPG
        pltpu.sync_copy(zero_v, sh_ea.at[pl.ds(s * TILE_SLICE, TILE_SLICE)])
        pltpu.sync_copy(zero_v, sh_cnt.at[pl.ds(s * TILE_SLICE, TILE_SLICE)])
        plsc.subcore_barrier()
        base = b * EPG + s * EPT
        pltpu.sync_copy(src_hbm.at[pl.ds(base, EPT)], src_v.at[pl.ds(0, EPT)])
        pltpu.sync_copy(dst_hbm.at[pl.ds(base, EPT)], dst_v.at[pl.ds(0, EPT)])
        pltpu.sync_copy(ea_hbm.at[pl.ds(base, EPT)], ea_v.at[pl.ds(0, EPT)])
        descs = []
        for r in range(8):
            def chunk(j, _, r=r):
                e0 = r * 128 + j * 16
                valid = (e0 + lane) < EPT
                s16 = src_v[pl.ds(e0, 16)] - off
                d16 = dst_v[pl.ds(e0, 16)] - off
                idx_v[r, pl.ds(j * 16, 16)] = jnp.where(valid, d16 * NP + s16, 0)
                vea_v[r, pl.ds(j * 16, 16)] = jnp.where(valid, ea_v[pl.ds(e0, 16)], 0.0)
                vcnt_v[r, pl.ds(j * 16, 16)] = jnp.where(valid, 1.0, 0.0)
                return 0
            lax.fori_loop(0, 8, chunk, 0)
            descs.append(pltpu.async_copy(vea_v.at[r], sh_ea.at[idx_v.at[r]], sem, add=True))
            descs.append(pltpu.async_copy(vcnt_v.at[r], sh_cnt.at[idx_v.at[r]], sem, add=True))
        for d in descs:
            d.wait()
        plsc.subcore_barrier()
        outs = []
        for k in range(32):
            row = s * 32 + k
            outs.append(pltpu.async_copy(sh_ea.at[pl.ds(row * NP, NP)],
                                         out_ea.at[b, row, :], sem))
            outs.append(pltpu.async_copy(sh_cnt.at[pl.ds(row * NP, NP)],
                                         out_cnt.at[b, row, :], sem))
        for d in outs:
            d.wait()
        return 0

    lax.fori_loop(0, GPC, per_graph, 0)


def _build_adj(src, dst, ea):
    mesh = plsc.VectorSubcoreMesh(core_axis_name="c", subcore_axis_name="s")
    fn = functools.partial(
        pl.kernel,
        out_type=[jax.ShapeDtypeStruct((B, NP, NP), jnp.float32),
                  jax.ShapeDtypeStruct((B, NP, NP), jnp.float32)],
        mesh=mesh,
        scratch_types=[
            pltpu.VMEM_SHARED((FLAT,), jnp.float32),
            pltpu.VMEM_SHARED((FLAT,), jnp.float32),
            pltpu.VMEM((1024,), jnp.int32),
            pltpu.VMEM((1024,), jnp.int32),
            pltpu.VMEM((1024,), jnp.float32),
            pltpu.VMEM((8, 128), jnp.int32),
            pltpu.VMEM((8, 128), jnp.float32),
            pltpu.VMEM((8, 128), jnp.float32),
            pltpu.VMEM((TILE_SLICE,), jnp.float32),
            pltpu.SemaphoreType.DMA,
        ],
    )(_sc_body)
    return fn(src, dst, ea)


def _gcn_pair_body(ate_ref, atc_ref, hc_ref, hp_ref,
                   ac_ref, cc_ref, wc_ref, bc_ref,
                   ap_ref, cp_ref, wp_ref, bp_ref,
                   oc_ref, op_ref):
    row_ok = jax.lax.broadcasted_iota(jnp.int32, (NP, 1), 0) < NPG

    def one_path(at, h, a, c, w, bias):
        deg = jnp.sum(at, axis=1, keepdims=True) + 1.0
        dinv = jax.lax.rsqrt(deg)
        hw = _dot(h * a + c, w, ((1,), (0,)))
        v = _dot(at, dinv * hw, ((1,), (0,)), _HI)
        out = dinv * v + (dinv * dinv) * hw + bias
        return jnp.where(row_ok, out, 0.0)

    for g in range(2):
        oc_ref[g] = one_path(ate_ref[g], hc_ref[g], ac_ref[...], cc_ref[...],
                             wc_ref[...], bc_ref[...])
        op_ref[g] = one_path(atc_ref[g], hp_ref[g], ap_ref[...], cp_ref[...],
                             wp_ref[...], bp_ref[...])


def _gcn_pair(ate, atc, hc, hp, ac, cc, wc, bc, ap, cp, wp, bp):
    co_c, co_p = wc.shape[1], wp.shape[1]
    blk3 = lambda d2, d3: pl.BlockSpec((2, d2, d3), lambda b: (b, 0, 0))
    full = lambda a: pl.BlockSpec(a.shape, lambda b: (0,) * a.ndim)
    args = (ate, atc, hc, hp, ac, cc, wc, bc, ap, cp, wp, bp)
    return pl.pallas_call(
        _gcn_pair_body,
        grid=(B // 2,),
        in_specs=[blk3(NP, NP), blk3(NP, NP), blk3(NP, hc.shape[2]), blk3(NP, hp.shape[2])]
                 + [full(a) for a in args[4:]],
        out_specs=[blk3(NP, co_c), blk3(NP, co_p)],
        out_shape=[jax.ShapeDtypeStruct((B, NP, co_c), jnp.float32),
                   jax.ShapeDtypeStruct((B, NP, co_p), jnp.float32)],
        interpret=_INTERP,
    )(*args)


def _diffpool_body(ate_ref, x11_ref, x12_ref, x13_ref, s11_ref, s12_ref, s13_ref,
                   wfc_ref, bfc_ref,
                   a11_ref, c11_ref, a12_ref, c12_ref, a13_ref, c13_ref,
                   b11_ref, d11_ref, b12_ref, d12_ref, b13_ref, d13_ref,
                   px_ref, padj_ref, x1o_ref):
    row_ok = jax.lax.broadcasted_iota(jnp.int32, (NP, 1), 0) < NPG
    s_cat = jnp.concatenate([s11_ref[0] * b11_ref[...] + d11_ref[...],
                             s12_ref[0] * b12_ref[...] + d12_ref[...],
                             s13_ref[0] * b13_ref[...] + d13_ref[...]], axis=1)
    s1 = _dot(s_cat, wfc_ref[...], ((1,), (0,))) + bfc_ref[...]
    mx = jnp.max(s1, axis=1, keepdims=True)
    e = jnp.exp(s1 - mx)
    s = e / jnp.sum(e, axis=1, keepdims=True)
    s = jnp.where(row_ok, s, 0.0)
    s_pad = jnp.concatenate([s, jnp.zeros((NP, KP - K), jnp.float32)], axis=1)

    x13bn = x13_ref[0] * a13_ref[...] + c13_ref[...]
    px_ref[0] = _dot(s_pad, x13bn, ((0,), (0,)))
    tmpd = _dot(ate_ref[0], s_pad, ((0,), (0,)))
    padj_ref[0] = _dot(s_pad, tmpd, ((0,), (0,)))

    x1cat = jnp.concatenate([x11_ref[0] * a11_ref[...] + c11_ref[...],
                             x12_ref[0] * a12_ref[...] + c12_ref[...],
                             x13bn], axis=1)
    x1cat = jnp.where(row_ok, x1cat, -1e30)
    x1o_ref[0, 0] = jnp.max(x1cat, axis=0)


def _diffpool(ate, x11, x12, x13, s11, s12, s13, wfc, bfc,
              a11, c11, a12, c12, a13, c13, b11, d11, b12, d12, b13, d13):
    blk3 = lambda d2, d3: pl.BlockSpec((1, d2, d3), lambda b: (b, 0, 0))
    full = lambda a: pl.BlockSpec(a.shape, lambda b: (0,) * a.ndim)
    args = (ate, x11, x12, x13, s11, s12, s13, wfc, bfc,
            a11, c11, a12, c12, a13, c13, b11, d11, b12, d12, b13, d13)
    return pl.pallas_call(
        _diffpool_body,
        grid=(B,),
        in_specs=[blk3(NP, NP)] + [blk3(NP, 30)] * 4 + [blk3(NP, 30), blk3(NP, K)]
                 + [full(a) for a in args[7:]],
        out_specs=[blk3(KP, 30), blk3(KP, KP), pl.BlockSpec((1, 1, 90), lambda b: (b, 0, 0))],
        out_shape=[jax.ShapeDtypeStruct((B, KP, 30), jnp.float32),
                   jax.ShapeDtypeStruct((B, KP, KP), jnp.float32),
                   jax.ShapeDtypeStruct((B, 1, 90), jnp.float32)],
        interpret=_INTERP,
    )(*args)


def _dense_gcn_body(adj_ref, h_ref, a_ref, c_ref, w_ref, b_ref, o_ref):
    row_ok = jax.lax.broadcasted_iota(jnp.int32, (KP, 1), 0) < K
    ri = jax.lax.broadcasted_iota(jnp.int32, (KP, KP), 0)
    ci = jax.lax.broadcasted_iota(jnp.int32, (KP, KP), 1)
    eye = jnp.where(ri == ci, 1.0, 0.0)
    adj = adj_ref[0]
    diag = jnp.sum(adj * eye, axis=0, keepdims=True)
    a2 = adj + eye * jnp.where(diag == 0.0, 1.0, 0.0)
    ones = jnp.ones((KP, 1), jnp.float32)
    deg = _dot(a2, ones, ((0,), (0,)), _HI)
    dinv = jnp.where(deg > 0.0, jax.lax.rsqrt(jnp.where(deg > 0.0, deg, 1.0)), 0.0)
    hw = _dot(h_ref[0] * a_ref[...] + c_ref[...], w_ref[...], ((1,), (0,)))
    v = _dot(a2, dinv * hw, ((0,), (0,)))
    o_ref[0] = jnp.where(row_ok, dinv * v + b_ref[...], 0.0)


def _dense_gcn(adj, h, a, c, w, bias):
    co = w.shape[1]
    blk3 = lambda d2, d3: pl.BlockSpec((1, d2, d3), lambda b: (b, 0, 0))
    full = lambda x: pl.BlockSpec(x.shape, lambda b: (0,) * x.ndim)
    return pl.pallas_call(
        _dense_gcn_body,
        grid=(B,),
        in_specs=[blk3(KP, KP), blk3(KP, h.shape[2]), full(a), full(c), full(w), full(bias)],
        out_specs=blk3(KP, co),
        out_shape=jax.ShapeDtypeStruct((B, KP, co), jnp.float32),
        interpret=_INTERP,
    )(adj, h, a, c, w, bias)


def _final_body(x1o_ref, x21_ref, x22_ref, x23_ref,
                a21_ref, c21_ref, a22_ref, c22_ref, a23_ref, c23_ref,
                w1_ref, b1_ref, w2_ref, b2_ref, o_ref):
    row_ok = jax.lax.broadcasted_iota(jnp.int32, (KP, 1), 0) < K
    x2cat = jnp.concatenate([x21_ref[0] * a21_ref[...] + c21_ref[...],
                             x22_ref[0] * a22_ref[...] + c22_ref[...],
                             x23_ref[0] * a23_ref[...] + c23_ref[...]], axis=1)
    x2cat = jnp.where(row_ok, x2cat, -1e30)
    x2row = jnp.max(x2cat, axis=0)
    conv = jnp.concatenate([x1o_ref[0, 0], x2row], axis=0).reshape(1, 180)
    h = jax.nn.relu(_dot(conv, w1_ref[...], ((1,), (0,))) + b1_ref[...])
    o_ref[0, 0] = (_dot(h, w2_ref[...], ((1,), (0,))) + b2_ref[...])[0]


def _final(x1o, x21, x22, x23, a21, c21, a22, c22, a23, c23, w1, b1, w2, b2):
    blk3 = lambda d2, d3: pl.BlockSpec((1, d2, d3), lambda b: (b, 0, 0))
    full = lambda a: pl.BlockSpec(a.shape, lambda b: (0,) * a.ndim)
    args = (x1o, x21, x22, x23, a21, c21, a22, c22, a23, c23, w1, b1, w2, b2)
    return pl.pallas_call(
        _final_body,
        grid=(B,),
        in_specs=[blk3(1, 90), blk3(KP, 30), blk3(KP, 30), blk3(KP, 30)]
                 + [full(a) for a in args[4:]],
        out_specs=pl.BlockSpec((1, 1, 6), lambda b: (b, 0, 0)),
        out_shape=jax.ShapeDtypeStruct((B, 1, 6), jnp.float32),
        interpret=_INTERP,
    )(*args)


def _stats_fold(h_pre, n_rows, bn):
    s1 = h_pre.sum(axis=(0, 1))
    s2 = (h_pre * h_pre).sum(axis=(0, 1))
    m = s1 / n_rows
    var = s2 / n_rows - m * m
    a = bn["g"] / jnp.sqrt(var + EPS)
    c = bn["be"] - m * a
    return a.reshape(1, -1), c.reshape(1, -1)


def kernel(x, edge_attr, params, edge_index):
    p = params
    src = edge_index[0].astype(jnp.int32)
    dst = edge_index[1].astype(jnp.int32)
    ate, atc = _build_adj(src, dst, edge_attr)

    xb = jnp.pad(x.reshape(B, NPG, 3), ((0, 0), (0, NP - NPG), (0, 5)))
    w1c = jnp.pad(p["conv11"]["W"], ((0, 5), (0, 0)))
    w1p = jnp.pad(p["pool_conv11"]["W"], ((0, 5), (0, 0)))
    one8 = jnp.ones((1, 8), jnp.float32)
    zero8 = jnp.zeros((1, 8), jnp.float32)
    one30 = jnp.ones((1, 30), jnp.float32)
    zero30 = jnp.zeros((1, 30), jnp.float32)
    rb = lambda v: v.reshape(1, -1)

    x11p, s11p = _gcn_pair(ate, atc, xb, xb,
                           one8, zero8, w1c, rb(p["conv11"]["b"]),
                           one8, zero8, w1p, rb(p["pool_conv11"]["b"]))
    a11, c11 = _stats_fold(x11p, N, p["norm11"])
    b11, d11 = _stats_fold(s11p, N, p["norm_p11"])
    x12p, s12p = _gcn_pair(ate, atc, x11p, s11p,
                           a11, c11, p["conv12"]["W"], rb(p["conv12"]["b"]),
                           b11, d11, p["pool_conv12"]["W"], rb(p["pool_conv12"]["b"]))
    a12, c12 = _stats_fold(x12p, N, p["norm12"])
    b12, d12 = _stats_fold(s12p, N, p["norm_p12"])
    x13p, s13p = _gcn_pair(ate, atc, x12p, s12p,
                           a12, c12, p["conv13"]["W"], rb(p["conv13"]["b"]),
                           b12, d12, p["pool_conv13"]["W"], rb(p["pool_conv13"]["b"]))
    a13, c13 = _stats_fold(x13p, N, p["norm13"])
    b13, d13 = _stats_fold(s13p, N, p["norm_p13"])

    p1x, p1adj, x1o = _diffpool(ate, x11p, x12p, x13p, s11p, s12p, s13p,
                                p["pool_fc"]["W"], rb(p["pool_fc"]["b"]),
                                a11, c11, a12, c12, a13, c13,
                                b11, d11, b12, d12, b13, d13)

    x21p = _dense_gcn(p1adj, p1x, one30, zero30, p["conv21"]["W"], rb(p["conv21"]["b"]))
    a21, c21 = _stats_fold(x21p, B * K, p["norm21"])
    x22p = _dense_gcn(p1adj, x21p, a21, c21, p["conv22"]["W"], rb(p["conv22"]["b"]))
    a22, c22 = _stats_fold(x22p, B * K, p["norm22"])
    x23p = _dense_gcn(p1adj, x22p, a22, c22, p["conv23"]["W"], rb(p["conv23"]["b"]))
    a23, c23 = _stats_fold(x23p, B * K, p["norm23"])

    out = _final(x1o, x21p, x22p, x23p, a21, c21, a22, c22, a23, c23,
                 p["fc1"]["W"], rb(p["fc1"]["b"]),
                 p["fc2"]["W"], rb(p["fc2"]["b"]))
    reg = jnp.zeros((1,), x.dtype)
    return (out.reshape(B, 6), reg)
